# baseline (speedup 1.0000x reference)
import functools

import jax
import jax.numpy as jnp
from jax import lax
from jax.experimental import pallas as pl
from jax.experimental.pallas import tpu as pltpu
from jax.experimental.pallas import tpu_sc as plsc

_NUM_SPECIAL = 4
_VOCAB = 1000000
_DIM = 64
_B_TOTAL = 4096 * 200
_NW = 32
_B_PER_W = _B_TOTAL // _NW
_CHUNK = 640
_K = _CHUNK // 128
_N_CHUNKS = _B_PER_W // _CHUNK
_L = 16


def _make_glove_body(b_per_w, n_chunks):
  def _glove_body(idx_hbm, special_hbm, pret_hbm, out_hbm,
                  idx0, idx1, pi0, pi1, rows0, rows1, special_v,
                  sem_g, sem_i, sem_w0, sem_w1):
    _B_PER_W = b_per_w
    _N_CHUNKS = n_chunks
    wid = lax.axis_index("c") * 16 + lax.axis_index("s")
    base_w = wid * _B_PER_W
    idx_b = (idx0, idx1)
    pi_b = (pi0, pi1)
    rows_b = (rows0, rows1)
    sem_w = (sem_w0, sem_w1)

    pltpu.sync_copy(special_hbm, special_v)
    pltpu.make_async_copy(
        idx_hbm.at[pl.ds(base_w, _CHUNK)], idx0, sem_i).start()

    def pair_body(ci2, carry):
        for b in (0, 1):
            c = ci2 * 2 + b
            base = base_w + c * _CHUNK
            idx_raw, pi_2d, rows = idx_b[b], pi_b[b], rows_b[b]

            pltpu.make_async_copy(
                idx_hbm.at[pl.ds(base, _CHUNK)], idx_raw, sem_i).wait()
            if b == 0:
                pltpu.make_async_copy(
                    idx_hbm.at[pl.ds(base + _CHUNK, _CHUNK)],
                    idx_b[1], sem_i).start()
            else:
                @pl.when(ci2 < _N_CHUNKS // 2 - 1)
                def _prefetch():
                    pltpu.make_async_copy(
                        idx_hbm.at[pl.ds(base + _CHUNK, _CHUNK)],
                        idx_b[0], sem_i).start()

            any_vec = jnp.zeros((_L,), jnp.int32)
            for i in range(_CHUNK // _L):
                v = idx_raw[pl.ds(i * _L, _L)]
                any_vec = any_vec | jnp.where(v < _NUM_SPECIAL, 1, 0).astype(jnp.int32)
                p = jnp.maximum(v - (_NUM_SPECIAL - 1), 0)
                p = jnp.minimum(p, _VOCAB + 1)
                pi_2d[i // 8, pl.ds((i % 8) * _L, _L)] = p
            n_special = any_vec[0]
            for t in range(1, _L):
                n_special = n_special | any_vec[t]

            @pl.when(ci2 >= 1)
            def _drain_prev(rows=rows, base=base, b=b):
                pltpu.make_async_copy(
                    rows,
                    out_hbm.at[pl.ds(base - 2 * _CHUNK, _CHUNK)],
                    sem_w[b]).wait()

            copies = []
            for j in range(_K):
                copies.append(pltpu.async_copy(
                    pret_hbm.at[pi_2d.at[j]],
                    rows.at[pl.ds(j * 128, 128)],
                    sem_g))
            for cc in copies:
                cc.wait()

            @pl.when(n_special > 0)
            def _fixup(idx_raw=idx_raw, rows=rows):
                def group_body(g, c2):
                    v = idx_raw[pl.ds(g * _L, _L)]
                    for j in range(_L):
                        ikj = v[j]

                        @pl.when(ikj < _NUM_SPECIAL)
                        def _fix(ikj=ikj, j=j):
                            row = g * _L + j
                            for cq in range(_DIM // _L):
                                sl = pl.ds(cq * _L, _L)
                                rows[row, sl] = rows[row, sl] + special_v[ikj, sl]
                    return c2

                lax.fori_loop(0, _CHUNK // _L, group_body, 0)

            pltpu.make_async_copy(
                rows, out_hbm.at[pl.ds(base, _CHUNK)], sem_w[b]).start()
        return carry

    lax.fori_loop(0, _N_CHUNKS // 2, pair_body, 0)

    for b in (0, 1):
        base = base_w + (_N_CHUNKS - 2 + b) * _CHUNK
        pltpu.make_async_copy(
            rows_b[b], out_hbm.at[pl.ds(base, _CHUNK)], sem_w[b]).wait()

  return _glove_body


_mesh = plsc.VectorSubcoreMesh(core_axis_name="c", subcore_axis_name="s")


def _make_glove_kernel(b_total):
    b_per_w = b_total // _NW
    n_chunks = b_per_w // _CHUNK
    return functools.partial(
        pl.kernel,
        mesh=_mesh,
        compiler_params=pltpu.CompilerParams(use_tc_tiling_on_sc=False),
        out_type=jax.ShapeDtypeStruct((b_total, _DIM), jnp.float32),
        scratch_types=[
            pltpu.VMEM((_CHUNK,), jnp.int32),
            pltpu.VMEM((_CHUNK,), jnp.int32),
            pltpu.VMEM((_K, 128), jnp.int32),
            pltpu.VMEM((_K, 128), jnp.int32),
            pltpu.VMEM((_CHUNK, _DIM), jnp.float32),
            pltpu.VMEM((_CHUNK, _DIM), jnp.float32),
            pltpu.VMEM((_NUM_SPECIAL + 1, _DIM), jnp.float32),
            pltpu.SemaphoreType.DMA,
            pltpu.SemaphoreType.DMA,
            pltpu.SemaphoreType.DMA,
            pltpu.SemaphoreType.DMA,
        ],
    )(_make_glove_body(b_per_w, n_chunks))


_glove_kernel = _make_glove_kernel(_B_TOTAL)


def _kernel_impl(indices, special_weight, pretrained_weights):
    flat = indices.reshape(-1)
    out = _glove_kernel(flat, special_weight, pretrained_weights)
    return out.reshape(indices.shape + (_DIM,))


kernel = jax.jit(_kernel_impl)
kernel.__name__ = "kernel"


# speedup vs baseline: 1.0012x; 1.0012x over previous
import functools

import jax
import jax.numpy as jnp
from jax import lax
from jax.experimental import pallas as pl
from jax.experimental.pallas import tpu as pltpu
from jax.experimental.pallas import tpu_sc as plsc

_NUM_SPECIAL = 4
_VOCAB = 1000000
_DIM = 64
_B_TOTAL = 4096 * 200
_NW = 32
_B_PER_W = _B_TOTAL // _NW
_CHUNK = 640
_K = _CHUNK // 128
_N_CHUNKS = _B_PER_W // _CHUNK
_L = 16


def _make_glove_body(b_per_w, n_chunks):
  def _glove_body(idx_hbm, special_hbm, pret_hbm, out_hbm,
                  idx0, idx1, pi0, pi1, rows0, rows1, special_v,
                  sem_g, sem_i, sem_w0, sem_w1):
    _B_PER_W = b_per_w
    _N_CHUNKS = n_chunks
    wid = lax.axis_index("c") * 16 + lax.axis_index("s")
    base_w = wid * _B_PER_W
    idx_b = (idx0, idx1)
    pi_b = (pi0, pi1)
    rows_b = (rows0, rows1)
    sem_w = (sem_w0, sem_w1)

    pltpu.sync_copy(special_hbm, special_v)

    sem_g_b = (sem_g, sem_i)

    def compute_pi(idx_raw, pi_2d):
        any_vec = jnp.zeros((_L,), jnp.int32)
        for i in range(_CHUNK // _L):
            v = idx_raw[pl.ds(i * _L, _L)]
            any_vec = any_vec | jnp.where(v < _NUM_SPECIAL, 1, 0).astype(jnp.int32)
            p = jnp.maximum(v - (_NUM_SPECIAL - 1), 0)
            p = jnp.minimum(p, _VOCAB + 1)
            pi_2d[i // 8, pl.ds((i % 8) * _L, _L)] = p
        ns = any_vec[0]
        for t in range(1, _L):
            ns = ns | any_vec[t]
        return ns

    def fire_gathers(pi_2d, rows, sg):
        for j in range(_K):
            pltpu.make_async_copy(
                pret_hbm.at[pi_2d.at[j]],
                rows.at[pl.ds(j * 128, 128)],
                sg).start()

    def drain_gathers(pi_2d, rows, sg):
        for j in range(_K):
            pltpu.make_async_copy(
                pret_hbm.at[pi_2d.at[j]],
                rows.at[pl.ds(j * 128, 128)],
                sg).wait()

    def fixup(n_special, idx_raw, rows):
        @pl.when(n_special > 0)
        def _fixup():
            def group_body(g, c2):
                v = idx_raw[pl.ds(g * _L, _L)]
                for j in range(_L):
                    ikj = v[j]

                    @pl.when(ikj < _NUM_SPECIAL)
                    def _fix(ikj=ikj, j=j):
                        row = g * _L + j
                        for cq in range(_DIM // _L):
                            sl = pl.ds(cq * _L, _L)
                            rows[row, sl] = rows[row, sl] + special_v[ikj, sl]
                return c2

            lax.fori_loop(0, _CHUNK // _L, group_body, 0)

    def wb_copy(c, b):
        return pltpu.make_async_copy(
            rows_b[b], out_hbm.at[pl.ds(base_w + c * _CHUNK, _CHUNK)],
            sem_w[b])

    pltpu.sync_copy(idx_hbm.at[pl.ds(base_w, _CHUNK)], idx0)
    ns_prev = compute_pi(idx0, pi0)
    fire_gathers(pi0, rows0, sem_g_b[0])

    def pair_body(ci2, ns_carry):
        ns_prev = ns_carry
        for b in (1, 0):
            c = ci2 * 2 + 1 + (1 - b)
            idx_raw, pi_2d, rows = idx_b[b], pi_b[b], rows_b[b]

            pltpu.sync_copy(
                idx_hbm.at[pl.ds(base_w + c * _CHUNK, _CHUNK)], idx_raw)
            ns_c = compute_pi(idx_raw, pi_2d)

            @pl.when(c >= 2)
            def _wb_drain(c=c, b=b):
                wb_copy(c - 2, b).wait()

            fire_gathers(pi_2d, rows, sem_g_b[b])

            drain_gathers(pi_b[1 - b], rows_b[1 - b], sem_g_b[1 - b])
            fixup(ns_prev, idx_b[1 - b], rows_b[1 - b])
            wb_copy(c - 1, 1 - b).start()
            ns_prev = ns_c
        return ns_prev

    ns_prev = lax.fori_loop(0, (_N_CHUNKS - 2) // 2, pair_body, ns_prev)

    c_last = _N_CHUNKS - 1
    pltpu.sync_copy(
        idx_hbm.at[pl.ds(base_w + c_last * _CHUNK, _CHUNK)], idx1)
    ns_last = compute_pi(idx1, pi1)
    wb_copy(c_last - 2, 1).wait()
    fire_gathers(pi1, rows1, sem_g_b[1])

    drain_gathers(pi0, rows0, sem_g_b[0])
    fixup(ns_prev, idx0, rows0)
    wb_copy(c_last - 1, 0).start()

    drain_gathers(pi1, rows1, sem_g_b[1])
    fixup(ns_last, idx1, rows1)
    wb_copy(c_last, 1).start()

    wb_copy(c_last - 1, 0).wait()
    wb_copy(c_last, 1).wait()

  return _glove_body


_mesh = plsc.VectorSubcoreMesh(core_axis_name="c", subcore_axis_name="s")


def _make_glove_kernel(b_total):
    b_per_w = b_total // _NW
    n_chunks = b_per_w // _CHUNK
    return functools.partial(
        pl.kernel,
        mesh=_mesh,
        compiler_params=pltpu.CompilerParams(use_tc_tiling_on_sc=False),
        out_type=jax.ShapeDtypeStruct((b_total, _DIM), jnp.float32),
        scratch_types=[
            pltpu.VMEM((_CHUNK,), jnp.int32),
            pltpu.VMEM((_CHUNK,), jnp.int32),
            pltpu.VMEM((_K, 128), jnp.int32),
            pltpu.VMEM((_K, 128), jnp.int32),
            pltpu.VMEM((_CHUNK, _DIM), jnp.float32),
            pltpu.VMEM((_CHUNK, _DIM), jnp.float32),
            pltpu.VMEM((_NUM_SPECIAL + 1, _DIM), jnp.float32),
            pltpu.SemaphoreType.DMA,
            pltpu.SemaphoreType.DMA,
            pltpu.SemaphoreType.DMA,
            pltpu.SemaphoreType.DMA,
        ],
    )(_make_glove_body(b_per_w, n_chunks))


_glove_kernel = _make_glove_kernel(_B_TOTAL)


def _kernel_impl(indices, special_weight, pretrained_weights):
    flat = indices.reshape(-1)
    out = _glove_kernel(flat, special_weight, pretrained_weights)
    return out.reshape(indices.shape + (_DIM,))


kernel = jax.jit(_kernel_impl)
kernel.__name__ = "kernel"
